# baseline (speedup 1.0000x reference)
import functools
import math

import jax
import jax.numpy as jnp
import numpy as np
from jax import lax
from jax.experimental import pallas as pl
from jax.experimental.pallas import tpu as pltpu
from jax.experimental.pallas import tpu_sc as plsc

_NUM_BUCKETS = 100
_EMBED_DIM = 128
_MAX_VAL = 1000000.0
_LOG_SCALE = (_NUM_BUCKETS - 1) / math.log(_MAX_VAL)
_I32_MAX = np.int32(2**31 - 1)
_I32_MIN = np.int32(-(2**31))

_BASES = np.round(
    np.exp(np.arange(_NUM_BUCKETS) * (math.log(_MAX_VAL) / (_NUM_BUCKETS - 1)))
).astype(np.int64)
_CANDS = (_BASES[:, None] + np.arange(-16, 16)[None, :]).astype(np.int32)


def _quantize_f32(t_i32):
    clamped = jnp.clip(t_i32.astype(jnp.float32), 1.0, None)
    log_times = jnp.log(clamped) / math.log(_MAX_VAL) * (_NUM_BUCKETS - 1)
    return jnp.clip(log_times.astype(jnp.int32), 0, _NUM_BUCKETS - 1)


def _thresholds():
    cands = lax.optimization_barrier(jnp.asarray(_CANDS))
    q = _quantize_f32(cands)
    ok = q >= jnp.arange(_NUM_BUCKETS, dtype=jnp.int32)[:, None]
    th = jnp.min(jnp.where(ok, cands, _I32_MAX), axis=1).astype(jnp.int32)
    return th.at[0].set(_I32_MIN)


def _project_tables_kernel(tabs_ref, w3_ref, b_ref, out_ref):
    for p in range(3):
        acc = lax.dot_general(
            tabs_ref[p], w3_ref[p], (((1,), (1,)), ((), ())),
            preferred_element_type=jnp.float32)
        if p == 0:
            acc = acc + b_ref[:]
        out_ref[p] = acc


def _projected_tables(abs_tab, rel_tab, sess_tab, W, b):
    d3 = _EMBED_DIM // 3

    def pad_tab(t):
        return jnp.pad(t, ((0, _EMBED_DIM - _NUM_BUCKETS), (0, 48 - t.shape[1])))

    tabs = jnp.stack([pad_tab(abs_tab), pad_tab(rel_tab), pad_tab(sess_tab)])
    w3 = jnp.stack([
        jnp.pad(W[:, 0:d3], ((0, 0), (0, 6))),
        jnp.pad(W[:, d3:2 * d3], ((0, 0), (0, 6))),
        jnp.pad(W[:, 2 * d3:], ((0, 0), (0, 4))),
    ])
    proj = pl.pallas_call(
        _project_tables_kernel,
        out_shape=jax.ShapeDtypeStruct((3, _EMBED_DIM, _EMBED_DIM), jnp.float32),
        in_specs=[
            pl.BlockSpec((3, _EMBED_DIM, 48), lambda: (0, 0, 0)),
            pl.BlockSpec((3, _EMBED_DIM, 48), lambda: (0, 0, 0)),
            pl.BlockSpec((1, _EMBED_DIM), lambda: (0, 0)),
        ],
        out_specs=pl.BlockSpec((3, _EMBED_DIM, _EMBED_DIM), lambda: (0, 0, 0)),
    )(tabs, w3, b.reshape(1, _EMBED_DIM))
    return proj.reshape(3 * _EMBED_DIM, _EMBED_DIM)


def _approx_bucket_kernel(ta_ref, tr_ref, ts_ref, ia_ref, ir_ref, is_ref):
    for p, (t_ref, o_ref) in enumerate(
            ((ta_ref, ia_ref), (tr_ref, ir_ref), (ts_ref, is_ref))):
        f = jnp.maximum(t_ref[:], 1).astype(jnp.float32)
        y = jnp.log(f) * np.float32(_LOG_SCALE)
        b0 = jnp.clip(y.astype(jnp.int32), 0, _NUM_BUCKETS - 1)
        o_ref[:] = b0 + np.int32(p * _EMBED_DIM)


def _approx_buckets(ta, tr, ts, T):
    rows = T // _EMBED_DIM
    br = 800
    grid = (rows // br,)
    spec = pl.BlockSpec((br, _EMBED_DIM), lambda i: (i, 0))
    sh = jax.ShapeDtypeStruct((rows, _EMBED_DIM), jnp.int32)
    r2 = lambda x: x.reshape(rows, _EMBED_DIM)
    ia, ir, is_ = pl.pallas_call(
        _approx_bucket_kernel,
        grid=grid,
        out_shape=(sh, sh, sh),
        in_specs=[spec, spec, spec],
        out_specs=(spec, spec, spec),
        compiler_params=pltpu.CompilerParams(
            dimension_semantics=("arbitrary",)),
    )(r2(ta), r2(tr), r2(ts))
    return ia.reshape(T), ir.reshape(T), is_.reshape(T)


_NC, _NS, _NW = 2, 16, 32
_CHUNK = 128


def _sc_body(p_hbm, ia_hbm, ir_hbm, is_hbm, ta_hbm, tr_hbm, ts_hbm, th_hbm,
             out_hbm, th_v, ia_v, ir_v, is_v, ta_v, tr_v, ts_v,
             bufa, bufr, bufs, sema, semr, sems):
    T = out_hbm.shape[0]
    tpw = T // _NW
    n_chunks = tpw // _CHUNK
    wid = lax.axis_index("s") * _NC + lax.axis_index("c")
    base = wid * tpw
    pltpu.sync_copy(th_hbm, th_v)

    def fix(idx_v, t_v):
        def vfix(v, _):
            sl = pl.ds(v * 16, 16)
            i = idx_v[sl]
            t = t_v[sl]
            lo = plsc.load_gather(th_v, [i])
            hi = plsc.load_gather(th_v, [i + 1])
            i = (i + (t >= hi).astype(jnp.int32)
                 - (t < lo).astype(jnp.int32))
            idx_v[sl] = i
            return 0

        lax.fori_loop(0, _CHUNK // 16, vfix, 0)

    def chunk(ci, _):
        off = base + ci * _CHUNK
        sl = pl.ds(off, _CHUNK)
        pltpu.sync_copy(ia_hbm.at[sl], ia_v)
        pltpu.sync_copy(ir_hbm.at[sl], ir_v)
        pltpu.sync_copy(is_hbm.at[sl], is_v)
        pltpu.sync_copy(ta_hbm.at[sl], ta_v)
        pltpu.sync_copy(tr_hbm.at[sl], tr_v)
        pltpu.sync_copy(ts_hbm.at[sl], ts_v)
        fix(ia_v, ta_v)
        fix(ir_v, tr_v)
        fix(is_v, ts_v)
        cpa = pltpu.async_copy(p_hbm.at[ia_v], bufa, sema)
        cpr = pltpu.async_copy(p_hbm.at[ir_v], bufr, semr)
        cps = pltpu.async_copy(p_hbm.at[is_v], bufs, sems)
        cpa.wait()
        cpr.wait()
        cps.wait()

        def vadd(t, _):
            for j in range(_EMBED_DIM // 16):
                s2 = pl.ds(j * 16, 16)
                bufa[t, s2] = bufa[t, s2] + bufr[t, s2] + bufs[t, s2]
            return 0

        lax.fori_loop(0, _CHUNK, vadd, 0)
        pltpu.sync_copy(bufa, out_hbm.at[sl])
        return 0

    lax.fori_loop(0, n_chunks, chunk, 0)


def kernel(timestamps, session_starts, abs_tab, rel_tab, sess_tab, W, b):
    B, L = timestamps.shape
    T = B * L

    ts = timestamps.astype(jnp.int32)
    t_rel = jnp.concatenate(
        [jnp.zeros((B, 1), jnp.int32), ts[:, 1:] - ts[:, :-1]], axis=1)
    t_sess = ts - session_starts.astype(jnp.int32)[:, None]
    ta, tr, tz = ts.reshape(T), t_rel.reshape(T), t_sess.reshape(T)

    p_flat = _projected_tables(abs_tab, rel_tab, sess_tab, W, b)
    ia0, ir0, is0 = _approx_buckets(ta, tr, tz, T)

    th = _thresholds()
    pad = jnp.full((_EMBED_DIM - _NUM_BUCKETS,), _I32_MAX, jnp.int32)
    blk = jnp.concatenate([th, pad])
    th3 = jnp.concatenate([blk, blk, blk, jnp.full((128,), _I32_MAX, jnp.int32)])

    sck = functools.partial(
        pl.kernel,
        out_type=jax.ShapeDtypeStruct((T, _EMBED_DIM), jnp.float32),
        mesh=plsc.VectorSubcoreMesh(core_axis_name="c", subcore_axis_name="s"),
        compiler_params=pltpu.CompilerParams(needs_layout_passes=False),
        scratch_types=[
            pltpu.VMEM((512,), jnp.int32),
            pltpu.VMEM((_CHUNK,), jnp.int32),
            pltpu.VMEM((_CHUNK,), jnp.int32),
            pltpu.VMEM((_CHUNK,), jnp.int32),
            pltpu.VMEM((_CHUNK,), jnp.int32),
            pltpu.VMEM((_CHUNK,), jnp.int32),
            pltpu.VMEM((_CHUNK,), jnp.int32),
            pltpu.VMEM((_CHUNK, _EMBED_DIM), jnp.float32),
            pltpu.VMEM((_CHUNK, _EMBED_DIM), jnp.float32),
            pltpu.VMEM((_CHUNK, _EMBED_DIM), jnp.float32),
            pltpu.SemaphoreType.DMA,
            pltpu.SemaphoreType.DMA,
            pltpu.SemaphoreType.DMA,
        ],
    )(_sc_body)
    out = sck(p_flat, ia0, ir0, is0, ta, tr, tz, th3)
    return out.reshape(B, L, _EMBED_DIM)
